# baseline (speedup 1.0000x reference)
import functools

import jax
import jax.numpy as jnp
from jax import lax
from jax.experimental import pallas as pl
from jax.experimental.pallas import tpu as pltpu
from jax.experimental.pallas import tpu_sc as plsc

VOCAB = 100000
EMBED = 128
BATCH = 1024
SEQ = 512
TYPE_VOCAB = 2

L = 16
NW = 32
N = BATCH * SEQ
ROWS_PER_W = N // NW
CHUNK = 256
NCHUNK = ROWS_PER_W // CHUNK
POS_PERIOD = SEQ // CHUNK


def _body(ids_hbm, tt_hbm, word_hbm, pos_hbm, ttab_hbm, out_hbm,
          idx_v, tt_idx_v, pos_idx_v, obuf_v, sem):
    wid = lax.axis_index("s") * 2 + lax.axis_index("c")
    wbase = wid * ROWS_PER_W

    for j in range(SEQ // L):
        pos_idx_v[pl.ds(j * L, L)] = lax.iota(jnp.int32, L) + (j * L)

    def chunk_step(c, _):
        base = wbase + c * CHUNK
        pos_off = (c % POS_PERIOD) * CHUNK

        pltpu.sync_copy(ids_hbm.at[pl.ds(base, CHUNK)], idx_v)
        pltpu.sync_copy(tt_hbm.at[pl.ds(base, CHUNK)], tt_idx_v)

        pltpu.async_copy(word_hbm.at[idx_v], obuf_v, sem).wait()
        pltpu.sync_copy(pos_hbm.at[pos_idx_v.at[pl.ds(pos_off, CHUNK)]],
                        obuf_v, add=True)
        pltpu.sync_copy(ttab_hbm.at[tt_idx_v], obuf_v, add=True)

        pltpu.sync_copy(obuf_v, out_hbm.at[pl.ds(base, CHUNK)])
        return _

    lax.fori_loop(0, NCHUNK, chunk_step, 0, unroll=False)


def kernel(input_ids, token_type_ids, word_emb, pos_emb, tok_type_emb):
    ids = input_ids.reshape(N).astype(jnp.int32)
    tt = token_type_ids.reshape(N).astype(jnp.int32)

    mesh = plsc.VectorSubcoreMesh(core_axis_name="c", subcore_axis_name="s")
    out = pl.kernel(
        _body,
        mesh=mesh,
        out_type=jax.ShapeDtypeStruct((N, EMBED), jnp.float32),
        scratch_types=[
            pltpu.VMEM((CHUNK,), jnp.int32),
            pltpu.VMEM((CHUNK,), jnp.int32),
            pltpu.VMEM((SEQ,), jnp.int32),
            pltpu.VMEM((CHUNK, EMBED), jnp.float32),
            pltpu.SemaphoreType.DMA,
        ],
    )(ids, tt, word_emb, pos_emb, tok_type_emb)
    return out.reshape(BATCH, SEQ, EMBED)
